# Initial kernel scaffold; baseline (speedup 1.0000x reference)
import jax
import jax.numpy as jnp
from jax.experimental import pallas as pl


def kernel(q, k, v, segment_ids, key_buffer, value_buffer, out_cache_loc):
    raise NotImplementedError("write your pallas kernel here")


# speedup vs baseline: 4.1847x; 4.1847x over previous
import functools

import jax
import jax.numpy as jnp
from jax import lax
from jax.experimental import pallas as pl
from jax.experimental.pallas import tpu as pltpu

NUM_HEADS = 16
HEAD_DIM = 128
SCALING = 0.08838834764831845
NEG = -1e30

BQ = 512
BK = 512


def _attn_kernel(seg_smem, q_ref, k_ref, v_ref, seg_row_ref, seg_col_ref, o_ref):
    i = pl.program_id(1)
    T = k_ref.shape[0]

    q = q_ref[...] * SCALING
    seg_q = seg_col_ref[...]

    target = seg_smem[i * BQ]

    def bs_body(_, lohi):
        lo, hi = lohi
        mid = (lo + hi) // 2
        pred = seg_smem[mid] < target
        lo = jnp.where(pred, mid + 1, lo)
        hi = jnp.where(pred, hi, mid)
        return lo, hi

    start, _ = lax.fori_loop(0, 13, bs_body, (jnp.int32(0), jnp.int32(T)))
    start_blk = start // BK

    rows = i * BQ + lax.broadcasted_iota(jnp.int32, (BQ, BK), 0)

    def inner(j, carry):
        m, l, acc = carry
        off = j * BK
        kc = k_ref[pl.ds(off, BK), :]
        vc = v_ref[pl.ds(off, BK), :]
        s = lax.dot_general(q, kc, (((1,), (1,)), ((), ())),
                            preferred_element_type=jnp.float32)
        seg_k = seg_row_ref[0:1, pl.ds(off, BK)]
        cols = off + lax.broadcasted_iota(jnp.int32, (BQ, BK), 1)
        mask = (seg_q == seg_k) & (rows >= cols)
        s = jnp.where(mask, s, NEG)
        m_new = jnp.maximum(m, jnp.max(s, axis=1, keepdims=True))
        alpha = jnp.exp(m - m_new)
        p = jnp.exp(s - m_new)
        l_new = l * alpha + jnp.sum(p, axis=1, keepdims=True)
        acc_new = acc * alpha + lax.dot_general(
            p, vc, (((1,), (0,)), ((), ())), preferred_element_type=jnp.float32)
        return m_new, l_new, acc_new

    m0 = jnp.full((BQ, 1), NEG, jnp.float32)
    l0 = jnp.zeros((BQ, 1), jnp.float32)
    acc0 = jnp.zeros((BQ, HEAD_DIM), jnp.float32)
    m, l, acc = lax.fori_loop(start_blk, i + 1, inner, (m0, l0, acc0))
    o_ref[...] = acc / l


def kernel(q, k, v, segment_ids, key_buffer, value_buffer, out_cache_loc):
    T = q.shape[0]
    nq = T // BQ
    seg = segment_ids.astype(jnp.int32)
    seg_row = seg.reshape(1, T)
    seg_col = seg.reshape(T, 1)

    grid_spec = pltpu.PrefetchScalarGridSpec(
        num_scalar_prefetch=1,
        grid=(NUM_HEADS, nq),
        in_specs=[
            pl.BlockSpec((BQ, HEAD_DIM), lambda h, i, seg_s: (i, h)),
            pl.BlockSpec((T, HEAD_DIM), lambda h, i, seg_s: (0, h)),
            pl.BlockSpec((T, HEAD_DIM), lambda h, i, seg_s: (0, h)),
            pl.BlockSpec((1, T), lambda h, i, seg_s: (0, 0)),
            pl.BlockSpec((BQ, 1), lambda h, i, seg_s: (i, 0)),
        ],
        out_specs=pl.BlockSpec((BQ, HEAD_DIM), lambda h, i, seg_s: (i, h)),
    )

    out = pl.pallas_call(
        _attn_kernel,
        grid_spec=grid_spec,
        out_shape=jax.ShapeDtypeStruct((T, NUM_HEADS * HEAD_DIM), jnp.float32),
        compiler_params=pltpu.CompilerParams(
            dimension_semantics=("arbitrary", "arbitrary"),
        ),
    )(seg, q, k, v, seg_row, seg_col)
    return out


# speedup vs baseline: 4.2168x; 1.0077x over previous
import functools

import jax
import jax.numpy as jnp
from jax import lax
from jax.experimental import pallas as pl
from jax.experimental.pallas import tpu as pltpu

NUM_HEADS = 16
HEAD_DIM = 128
SCALING = 0.08838834764831845
NEG = -1e30

BQ = 512
BK = 512


def _attn_kernel(seg_smem, q_ref, k_ref, v_ref, seg_row_ref, seg_col_ref, o_ref):
    i = pl.program_id(1)
    T = k_ref.shape[0]

    q = q_ref[...] * SCALING
    seg_q = seg_col_ref[...]

    target = seg_smem[i * BQ]

    def bs_body(_, lohi):
        lo, hi = lohi
        mid = (lo + hi) // 2
        pred = seg_smem[mid] < target
        lo = jnp.where(pred, mid + 1, lo)
        hi = jnp.where(pred, hi, mid)
        return lo, hi

    start, _ = lax.fori_loop(0, 13, bs_body, (jnp.int32(0), jnp.int32(T)))
    start_blk = start // BK

    rows = i * BQ + lax.broadcasted_iota(jnp.int32, (BQ, BK), 0)

    def inner(j, carry):
        m, l, acc = carry
        off = j * BK
        kc = k_ref[pl.ds(off, BK), :]
        vc = v_ref[pl.ds(off, BK), :]
        s = lax.dot_general(q, kc, (((1,), (1,)), ((), ())),
                            preferred_element_type=jnp.float32)
        seg_k = seg_row_ref[0:1, pl.ds(off, BK)]
        cols = off + lax.broadcasted_iota(jnp.int32, (BQ, BK), 1)
        mask = (seg_q == seg_k) & (rows >= cols)
        s = jnp.where(mask, s, NEG)
        m_new = jnp.maximum(m, jnp.max(s, axis=1, keepdims=True))
        alpha = jnp.exp(m - m_new)
        p = jnp.exp(s - m_new)
        l_new = l * alpha + jnp.sum(p, axis=1, keepdims=True)
        acc_new = acc * alpha + lax.dot_general(
            p, vc, (((1,), (0,)), ((), ())), preferred_element_type=jnp.float32)
        return m_new, l_new, acc_new

    m0 = jnp.full((BQ, 1), NEG, jnp.float32)
    l0 = jnp.zeros((BQ, 1), jnp.float32)
    acc0 = jnp.zeros((BQ, HEAD_DIM), jnp.float32)
    m, l, acc = lax.fori_loop(start_blk, i + 1, inner, (m0, l0, acc0))
    o_ref[...] = acc / l


def kernel(q, k, v, segment_ids, key_buffer, value_buffer, out_cache_loc):
    T = q.shape[0]
    nq = T // BQ
    seg = segment_ids.astype(jnp.int32)
    seg_row = seg.reshape(1, T)
    seg_col = seg.reshape(T, 1)

    grid_spec = pltpu.PrefetchScalarGridSpec(
        num_scalar_prefetch=1,
        grid=(NUM_HEADS, nq),
        in_specs=[
            pl.BlockSpec((BQ, HEAD_DIM), lambda h, i, seg_s: (i, h)),
            pl.BlockSpec((T, HEAD_DIM), lambda h, i, seg_s: (0, h)),
            pl.BlockSpec((T, HEAD_DIM), lambda h, i, seg_s: (0, h)),
            pl.BlockSpec((1, T), lambda h, i, seg_s: (0, 0)),
            pl.BlockSpec((BQ, 1), lambda h, i, seg_s: (i, 0)),
        ],
        out_specs=pl.BlockSpec((BQ, HEAD_DIM), lambda h, i, seg_s: (i, h)),
    )

    out = pl.pallas_call(
        _attn_kernel,
        grid_spec=grid_spec,
        out_shape=jax.ShapeDtypeStruct((T, NUM_HEADS * HEAD_DIM), jnp.float32),
        compiler_params=pltpu.CompilerParams(
            dimension_semantics=("parallel", "arbitrary"),
        ),
    )(seg, q, k, v, seg_row, seg_col)
    return out
